# baseline (speedup 1.0000x reference)
import functools

import jax
import jax.numpy as jnp
from jax import lax
from jax.experimental import pallas as pl
from jax.experimental.pallas import tpu as pltpu
from jax.experimental.pallas import tpu_sc as plsc

N, D, M = 100000, 128, 64
NC, NS = 2, 16
NW = NC * NS
CH = 128
FULL = N // CH
TAIL = N - FULL * CH
STEPS = (FULL + 1 + NW - 1) // NW

_SC_MESH = plsc.VectorSubcoreMesh(core_axis_name="c", subcore_axis_name="s")
_LANES = 16


@functools.partial(
    pl.kernel,
    out_type=jax.ShapeDtypeStruct((NC, M, D), jnp.float32),
    mesh=_SC_MESH,
    scratch_types=[
        pltpu.VMEM((CH,), jnp.int32),
        pltpu.VMEM((CH, D), jnp.float32),
        pltpu.VMEM((TAIL,), jnp.int32),
        pltpu.VMEM((TAIL, D), jnp.float32),
        pltpu.VMEM_SHARED((M, D), jnp.float32),
    ],
)
def _pool_kernel(x_hbm, batch_hbm, zeros_hbm, out_hbm,
                 idx_v, xbuf, idx_t, xtail, pool_sh):
    cid = lax.axis_index("c")
    sid = lax.axis_index("s")
    wid = sid * NC + cid

    @pl.when(sid == 0)
    def _():
        pltpu.sync_copy(zeros_hbm, pool_sh)

    plsc.subcore_barrier()

    for i in range(STEPS):
        c = wid + i * NW

        @pl.when(c < FULL)
        def _():
            base = c * CH
            pltpu.sync_copy(batch_hbm.at[pl.ds(base, CH)], idx_v)
            pltpu.sync_copy(x_hbm.at[pl.ds(base, CH)], xbuf)
            pltpu.sync_copy(xbuf, pool_sh.at[idx_v], add=True)

        @pl.when(c == FULL)
        def _():
            pltpu.sync_copy(batch_hbm.at[pl.ds(FULL * CH, TAIL)], idx_t)
            pltpu.sync_copy(x_hbm.at[pl.ds(FULL * CH, TAIL)], xtail)
            pltpu.sync_copy(xtail, pool_sh.at[idx_t], add=True)

    plsc.subcore_barrier()

    @pl.when(sid == 0)
    def _():
        pltpu.sync_copy(pool_sh, out_hbm.at[cid])


def _mlp_body(pp_ref, vn_ref, w_ref, b_ref, out_ref):
    pool = pp_ref[0] + pp_ref[1]
    h = jnp.dot(vn_ref[:] + pool, w_ref[:],
                preferred_element_type=jnp.float32) + b_ref[:]
    out_ref[:] = vn_ref[:] + jnp.maximum(h, 0.0)


_mlp = pl.pallas_call(
    _mlp_body,
    out_shape=jax.ShapeDtypeStruct((M, D), jnp.float32),
)


@functools.partial(
    pl.kernel,
    out_type=jax.ShapeDtypeStruct((N, D), jnp.float32),
    mesh=_SC_MESH,
    scratch_types=[
        pltpu.VMEM((CH,), jnp.int32),
        pltpu.VMEM((CH, D), jnp.float32),
        pltpu.VMEM((CH, D), jnp.float32),
        pltpu.VMEM((TAIL,), jnp.int32),
        pltpu.VMEM((TAIL, D), jnp.float32),
        pltpu.VMEM((TAIL, D), jnp.float32),
    ],
)
def _bcast_kernel(x_hbm, batch_hbm, vn_hbm, out_hbm,
                  idx_v, xbuf, vbuf, idx_t, xtail, vtail):
    cid = lax.axis_index("c")
    sid = lax.axis_index("s")
    wid = sid * NC + cid

    def _add_rows(dst, src, rows):
        def body(r, carry):
            for j in range(D // _LANES):
                s = pl.ds(j * _LANES, _LANES)
                dst[r, s] = dst[r, s] + src[r, s]
            return carry
        lax.fori_loop(0, rows, body, 0)

    for i in range(STEPS):
        c = wid + i * NW

        @pl.when(c < FULL)
        def _():
            base = c * CH
            pltpu.sync_copy(batch_hbm.at[pl.ds(base, CH)], idx_v)
            pltpu.sync_copy(x_hbm.at[pl.ds(base, CH)], xbuf)
            pltpu.sync_copy(vn_hbm.at[idx_v], vbuf)
            _add_rows(xbuf, vbuf, CH)
            pltpu.sync_copy(xbuf, out_hbm.at[pl.ds(base, CH)])

        @pl.when(c == FULL)
        def _():
            base = FULL * CH
            pltpu.sync_copy(batch_hbm.at[pl.ds(base, TAIL)], idx_t)
            pltpu.sync_copy(x_hbm.at[pl.ds(base, TAIL)], xtail)
            pltpu.sync_copy(vn_hbm.at[idx_t], vtail)
            _add_rows(xtail, vtail, TAIL)
            pltpu.sync_copy(xtail, out_hbm.at[pl.ds(base, TAIL)])


def kernel(x, vn_x, batch, W, b):
    zeros = jnp.zeros((M, D), jnp.float32)
    pp = _pool_kernel(x, batch, zeros)
    vn_out = _mlp(pp, vn_x, W, b.reshape(1, D))
    x_out = _bcast_kernel(x, batch, vn_out)
    return (x_out, vn_out)


# speedup vs baseline: 3.9596x; 3.9596x over previous
import functools

import jax
import jax.numpy as jnp
from jax import lax
from jax.experimental import pallas as pl
from jax.experimental.pallas import tpu as pltpu
from jax.experimental.pallas import tpu_sc as plsc

N, D, M = 100000, 128, 64
NC, NS = 2, 16
NW = NC * NS
CH = 128
FULL = N // CH
TAIL = N - FULL * CH
STEPS = (FULL + 1 + NW - 1) // NW
L = 16
GPC = CH // L
JB = D // L

_SC_MESH = plsc.VectorSubcoreMesh(core_axis_name="c", subcore_axis_name="s")
_SC_PARAMS = pltpu.CompilerParams(needs_layout_passes=False)


def _lane_eq(r):
    return lax.iota(jnp.int32, L) == r


def _row_seg(segs, r):
    return jnp.max(jnp.where(_lane_eq(r), segs, jnp.int32(-1)))


def _accumulate_groups(idx, xf, accv, ngroups):
    def group_body(g, carry):
        segs = idx[pl.ds(g * L, L)]
        smin = jnp.min(segs)
        smax = jnp.max(segs)

        def uniform(_):
            acc = [xf[g * L, pl.ds(j * L, L)] for j in range(JB)]
            for r in range(1, L):
                for j in range(JB):
                    acc[j] = acc[j] + xf[g * L + r, pl.ds(j * L, L)]
            for j in range(JB):
                s = pl.ds(j * L, L)
                accv[smin, s] = accv[smin, s] + acc[j]
            return 0

        def boundary(_):
            for r in range(L):
                seg = _row_seg(segs, r)
                for j in range(JB):
                    s = pl.ds(j * L, L)
                    accv[seg, s] = accv[seg, s] + xf[g * L + r, s]
            return 0

        lax.cond(smin == smax, uniform, boundary, 0)
        return carry

    lax.fori_loop(0, ngroups, group_body, 0)


@functools.partial(
    pl.kernel,
    out_type=jax.ShapeDtypeStruct((NW, M, D), jnp.float32),
    mesh=_SC_MESH,
    compiler_params=_SC_PARAMS,
    scratch_types=[
        pltpu.VMEM((CH,), jnp.int32),
        pltpu.VMEM((CH, D), jnp.float32),
        pltpu.VMEM((M, D), jnp.float32),
    ],
)
def _pool_kernel(x_hbm, batch_hbm, out_hbm, idx, xf, accv):
    cid = lax.axis_index("c")
    sid = lax.axis_index("s")
    wid = sid * NC + cid

    zero = jnp.zeros((L,), jnp.float32)

    def zero_body(m, carry):
        for j in range(JB):
            accv[m, pl.ds(j * L, L)] = zero
        return carry

    lax.fori_loop(0, M, zero_body, 0)

    def step(i, carry):
        c = wid + i * NW

        @pl.when(c < FULL)
        def _():
            base = c * CH
            pltpu.sync_copy(batch_hbm.at[pl.ds(base, CH)], idx)
            pltpu.sync_copy(x_hbm.at[pl.ds(base, CH)], xf)
            _accumulate_groups(idx, xf, accv, GPC)

        @pl.when(c == FULL)
        def _():
            base = FULL * CH
            pltpu.sync_copy(batch_hbm.at[pl.ds(base, TAIL)],
                            idx.at[pl.ds(0, TAIL)])
            pltpu.sync_copy(x_hbm.at[pl.ds(base, TAIL)],
                            xf.at[pl.ds(0, TAIL)])
            _accumulate_groups(idx, xf, accv, TAIL // L)

        return carry

    lax.fori_loop(0, STEPS, step, 0)
    pltpu.sync_copy(accv, out_hbm.at[wid])


def _mlp_body(pp_ref, vn_ref, w_ref, b_ref, out_ref):
    pool = jnp.sum(pp_ref[:], axis=0)
    h = jnp.dot(vn_ref[:] + pool, w_ref[:],
                preferred_element_type=jnp.float32) + b_ref[:]
    out_ref[:] = vn_ref[:] + jnp.maximum(h, 0.0)


_mlp = pl.pallas_call(
    _mlp_body,
    out_shape=jax.ShapeDtypeStruct((M, D), jnp.float32),
)


def _broadcast_groups(idx, xf, vn, ngroups):
    def group_body(g, carry):
        segs = idx[pl.ds(g * L, L)]
        smin = jnp.min(segs)
        smax = jnp.max(segs)

        def uniform(_):
            vr = [vn[smin, pl.ds(j * L, L)] for j in range(JB)]
            for r in range(L):
                for j in range(JB):
                    s = pl.ds(j * L, L)
                    xf[g * L + r, s] = xf[g * L + r, s] + vr[j]
            return 0

        def boundary(_):
            for r in range(L):
                seg = _row_seg(segs, r)
                for j in range(JB):
                    s = pl.ds(j * L, L)
                    xf[g * L + r, s] = xf[g * L + r, s] + vn[seg, s]
            return 0

        lax.cond(smin == smax, uniform, boundary, 0)
        return carry

    lax.fori_loop(0, ngroups, group_body, 0)


@functools.partial(
    pl.kernel,
    out_type=jax.ShapeDtypeStruct((N, D), jnp.float32),
    mesh=_SC_MESH,
    compiler_params=_SC_PARAMS,
    scratch_types=[
        pltpu.VMEM((CH,), jnp.int32),
        pltpu.VMEM((CH, D), jnp.float32),
        pltpu.VMEM((M, D), jnp.float32),
    ],
)
def _bcast_kernel(x_hbm, batch_hbm, vn_hbm, out_hbm, idx, xf, vn):
    cid = lax.axis_index("c")
    sid = lax.axis_index("s")
    wid = sid * NC + cid

    pltpu.sync_copy(vn_hbm, vn)

    def step(i, carry):
        c = wid + i * NW

        @pl.when(c < FULL)
        def _():
            base = c * CH
            pltpu.sync_copy(batch_hbm.at[pl.ds(base, CH)], idx)
            pltpu.sync_copy(x_hbm.at[pl.ds(base, CH)], xf)
            _broadcast_groups(idx, xf, vn, GPC)
            pltpu.sync_copy(xf, out_hbm.at[pl.ds(base, CH)])

        @pl.when(c == FULL)
        def _():
            base = FULL * CH
            pltpu.sync_copy(batch_hbm.at[pl.ds(base, TAIL)],
                            idx.at[pl.ds(0, TAIL)])
            pltpu.sync_copy(x_hbm.at[pl.ds(base, TAIL)],
                            xf.at[pl.ds(0, TAIL)])
            _broadcast_groups(idx, xf, vn, TAIL // L)
            pltpu.sync_copy(xf.at[pl.ds(0, TAIL)], out_hbm.at[pl.ds(base, TAIL)])

        return carry

    lax.fori_loop(0, STEPS, step, 0)


def kernel(x, vn_x, batch, W, b):
    pp = _pool_kernel(x, batch)
    vn_out = _mlp(pp, vn_x, W, b.reshape(1, D))
    x_out = _bcast_kernel(x, batch, vn_out)
    return (x_out, vn_out)
